# baseline (speedup 1.0000x reference)
import jax
import jax.numpy as jnp
from jax import lax
from jax.experimental import pallas as pl
from jax.experimental.pallas import tpu as pltpu
from jax.experimental.pallas import tpu_sc as plsc

B, L, H = 1024, 200, 128
NW = 32
RPW = B * L // NW
CH = 128
NCH = RPW // CH
NBUF = 3
LANES = 16
VPR = H // LANES


def _body(seq_hbm, word_hbm, pos_hbm, out_hbm, idx_v, pos2_v, bufs_v, gsem, osem):
    wid = lax.axis_index("s") * 2 + lax.axis_index("c")
    base = wid * RPW

    pltpu.sync_copy(seq_hbm.at[pl.ds(base, RPW)], idx_v)
    pltpu.sync_copy(pos_hbm, pos2_v.at[pl.ds(0, L)])
    pltpu.sync_copy(pos_hbm, pos2_v.at[pl.ds(L, L)])

    def start_gather(c, slot):
        pltpu.async_copy(
            word_hbm.at[idx_v.at[pl.ds(c * CH, CH)]],
            bufs_v.at[slot],
            gsem.at[slot],
        )

    start_gather(0, 0)
    start_gather(1, 1)

    def chunk_loop(c, carry):
        p = lax.rem(c, NBUF)
        pltpu.make_async_copy(
            word_hbm.at[idx_v.at[pl.ds(c * CH, CH)]],
            bufs_v.at[p],
            gsem.at[p],
        ).wait()

        p0 = lax.rem(base + c * CH, L)

        def row_loop(r, cc):
            for k in range(VPR):
                sl = pl.ds(k * LANES, LANES)
                bufs_v[p, r, sl] = jnp.maximum(
                    bufs_v[p, r, sl] + pos2_v[p0 + r, sl], 0.0
                )
            return cc

        lax.fori_loop(0, CH, row_loop, 0)

        pltpu.async_copy(
            bufs_v.at[p],
            out_hbm.at[pl.ds(base + c * CH, CH)],
            osem.at[p],
        )

        p2 = lax.rem(c + 2, NBUF)

        @pl.when(c >= 1)
        def _():
            pltpu.make_async_copy(
                bufs_v.at[p2],
                out_hbm.at[pl.ds(base, CH)],
                osem.at[p2],
            ).wait()

        @pl.when(c < NCH - 2)
        def _():
            start_gather(c + 2, p2)

        return carry

    lax.fori_loop(0, NCH, chunk_loop, 0)

    last = (NCH - 1) % NBUF
    pltpu.make_async_copy(
        bufs_v.at[last],
        out_hbm.at[pl.ds(base, CH)],
        osem.at[last],
    ).wait()


def kernel(input_seq, word_table, pos_table):
    seq = input_seq.astype(jnp.int32).reshape(B * L)
    mesh = plsc.VectorSubcoreMesh(core_axis_name="c", subcore_axis_name="s")
    f = pl.kernel(
        _body,
        mesh=mesh,
        out_type=jax.ShapeDtypeStruct((B * L, H), jnp.float32),
        scratch_types=[
            pltpu.VMEM((RPW,), jnp.int32),
            pltpu.VMEM((2 * L, H), jnp.float32),
            pltpu.VMEM((NBUF, CH, H), jnp.float32),
            pltpu.SemaphoreType.DMA((NBUF,)),
            pltpu.SemaphoreType.DMA((NBUF,)),
        ],
    )
    return f(seq, word_table, pos_table).reshape(B, L, H)
